# baseline (speedup 1.0000x reference)
import functools

import numpy as np
import jax
import jax.numpy as jnp
from jax import lax
from jax.experimental import pallas as pl
from jax.experimental.pallas import tpu as pltpu

_FACTOR = 5


def _threefry2x32(ks, c1, c2):
    def rotl(x, d):
        return ((x << np.uint32(d)) | (x >> np.uint32(32 - d))).astype(np.uint32)
    ks0, ks1 = np.uint32(ks[0]), np.uint32(ks[1])
    ks2 = np.uint32(ks0 ^ ks1 ^ np.uint32(0x1BD11BDA))
    x0 = (c1 + ks0).astype(np.uint32)
    x1 = (c2 + ks1).astype(np.uint32)
    rot = [[13, 15, 26, 6], [17, 29, 16, 24]]
    sched = [(ks1, ks2), (ks2, ks0), (ks0, ks1), (ks1, ks2), (ks2, ks0)]
    for i in range(5):
        for r in rot[i % 2]:
            x0 = (x0 + x1).astype(np.uint32)
            x1 = rotl(x1, r)
            x1 = (x1 ^ x0).astype(np.uint32)
        a, b = sched[i]
        x0 = (x0 + a).astype(np.uint32)
        x1 = (x1 + b + np.uint32(i + 1)).astype(np.uint32)
    return x0, x1


def _random_bits(ks, n):
    x0, x1 = _threefry2x32(ks, np.zeros(n, np.uint32), np.arange(n, dtype=np.uint32))
    return (x0 ^ x1).astype(np.uint32)


def _sample_indices(L, sample_k, seed=42):
    key = (np.uint32(0), np.uint32(seed))
    x0, x1 = _threefry2x32(key, np.zeros(2, np.uint32), np.arange(2, dtype=np.uint32))
    k1, k2 = (x0[0], x1[0]), (x0[1], x1[1])
    n = L * sample_k
    hi, lo = _random_bits(k1, n), _random_bits(k2, n)
    span = np.uint32(L)
    mult = np.uint32((np.uint64(2) ** 16 % span) ** 2 % span)
    off = ((hi % span) * mult + lo % span) % span
    return off.astype(np.int32).reshape(L, sample_k)


@functools.lru_cache(maxsize=4)
def _constants(L, sample_k, CH):
    idx = _sample_indices(L, sample_k)
    AT = np.zeros((L, L), np.int32)
    np.add.at(AT, (idx, np.arange(L)[:, None]), 1)
    NEG = np.where(AT > 0, 0.0, -1e38).astype(np.float32)
    TRI = np.tril(np.ones((CH, CH), np.float32))
    return AT.astype(np.float32), NEG, TRI


def _east_body(x_ref, ste_ref, wq_ref, bq_ref, wk_ref, bk_ref, wv_ref, bv_ref,
               at_ref, neg_ref, tri_ref, out_ref, *, L, D, n_top, U, CH):
    f32 = jnp.float32
    x = x_ref[0, 0]
    s = ste_ref[0, 0]
    dn11 = (((1,), (1,)), ((), ()))
    dn10 = (((1,), (0,)), ((), ()))
    dn00 = (((0,), (0,)), ((), ()))

    def proj(w_ref, b_ref):
        w = w_ref[...]
        h = (lax.dot_general(x, w[:, :D], dn11)
             + lax.dot_general(s, w[:, D:], dn11)
             + b_ref[...])
        return jnp.maximum(h, 0.0)

    Qm = proj(wq_ref, bq_ref)
    Km = proj(wk_ref, bk_ref)
    Vm = proj(wv_ref, bv_ref)

    ST = lax.dot_general(Km, Qm, dn11)
    atf = at_ref[...]
    smax = jnp.max(ST + neg_ref[...], axis=0, keepdims=True)
    W = lax.dot_general(atf, Km, dn00)
    ssum_col = lax.dot_general(Qm * W, jnp.ones((D, 1), f32), dn10)
    ssum_row = jnp.transpose(ssum_col)
    Mrow = smax - ssum_row * (1.0 / L)
    Mcol = jnp.transpose(Mrow)

    ii = lax.broadcasted_iota(jnp.int32, (L, L), 0)
    jj = lax.broadcasted_iota(jnp.int32, (L, L), 1)
    beats = (Mcol > Mrow) | ((Mcol == Mrow) & (ii < jj))
    C = beats.astype(jnp.bfloat16)
    rank = lax.dot_general(jnp.ones((1, L), jnp.bfloat16), C, dn10,
                           preferred_element_type=f32)

    iota_uf = lax.broadcasted_iota(jnp.int32, (U, 1), 0).astype(f32)
    P = ((rank == iota_uf) & (iota_uf < n_top)).astype(f32)
    iota_cf = lax.broadcasted_iota(jnp.int32, (L, 1), 0).astype(f32)
    mt = lax.dot_general(P, iota_cf, dn10)

    lane = lax.broadcasted_iota(jnp.int32, (U, L), 1).astype(f32)
    Qr = lax.dot_general(P, Qm, dn10)
    sc = lax.dot_general(Qr, Km, dn11) * (1.0 / np.sqrt(D))
    sc = jnp.where(lane > mt, -1e30, sc)
    sc = sc - jnp.max(sc, axis=1, keepdims=True)
    e = jnp.exp(sc)
    attn = e / jnp.sum(e, axis=1, keepdims=True)
    upd = lax.dot_general(attn, Vm, dn10)

    tri = tri_ref[...]
    carry = jnp.zeros((1, D), f32)
    rows = []
    for i in range(L // CH):
        vi = lax.slice(Vm, (i * CH, 0), ((i + 1) * CH, D))
        rows.append(lax.dot_general(tri, vi, dn10) + carry)
        carry = carry + lax.dot_general(jnp.ones((1, CH), f32), vi, dn10)
    ctx = jnp.concatenate(rows, axis=0)

    scat = lax.dot_general(P, upd, dn00)
    selc = lax.dot_general(P, jnp.ones((U, 1), f32), dn00)
    out_ref[0] = ctx * (1.0 - selc) + scat


def kernel(X, STE, Wq, bq, Wk, bk, Wv, bv, attn_mask=0):
    B, H, L, D = X.shape
    n_top = min(_FACTOR * int(np.ceil(np.log(L))), L)
    U = ((n_top + 7) // 8) * 8
    CH = 128
    ATF, NEG, TRI = _constants(L, n_top, CH)

    out = pl.pallas_call(
        functools.partial(_east_body, L=L, D=D, n_top=n_top, U=U, CH=CH),
        grid=(B, H),
        in_specs=[
            pl.BlockSpec((1, 1, L, D), lambda b, h: (b, h, 0, 0)),
            pl.BlockSpec((1, 1, L, D), lambda b, h: (b, h, 0, 0)),
            pl.BlockSpec((D, 2 * D), lambda b, h: (0, 0)),
            pl.BlockSpec((1, D), lambda b, h: (0, 0)),
            pl.BlockSpec((D, 2 * D), lambda b, h: (0, 0)),
            pl.BlockSpec((1, D), lambda b, h: (0, 0)),
            pl.BlockSpec((D, 2 * D), lambda b, h: (0, 0)),
            pl.BlockSpec((1, D), lambda b, h: (0, 0)),
            pl.BlockSpec((L, L), lambda b, h: (0, 0)),
            pl.BlockSpec((L, L), lambda b, h: (0, 0)),
            pl.BlockSpec((CH, CH), lambda b, h: (0, 0)),
        ],
        out_specs=pl.BlockSpec((1, L, D), lambda b, h: (b, 0, h)),
        out_shape=jax.ShapeDtypeStruct((B, L, H * D), jnp.float32),
        compiler_params=pltpu.CompilerParams(
            dimension_semantics=("parallel", "parallel")),
    )(X, STE, Wq, bq.reshape(1, D), Wk, bk.reshape(1, D), Wv, bv.reshape(1, D),
      jnp.asarray(ATF), jnp.asarray(NEG), jnp.asarray(TRI))
    return out.reshape(B, L, H, D)


# speedup vs baseline: 1.0369x; 1.0369x over previous
import functools

import numpy as np
import jax
import jax.numpy as jnp
from jax import lax
from jax.experimental import pallas as pl
from jax.experimental.pallas import tpu as pltpu

_FACTOR = 5


def _threefry2x32(ks, c1, c2):
    def rotl(x, d):
        return ((x << np.uint32(d)) | (x >> np.uint32(32 - d))).astype(np.uint32)
    ks0, ks1 = np.uint32(ks[0]), np.uint32(ks[1])
    ks2 = np.uint32(ks0 ^ ks1 ^ np.uint32(0x1BD11BDA))
    x0 = (c1 + ks0).astype(np.uint32)
    x1 = (c2 + ks1).astype(np.uint32)
    rot = [[13, 15, 26, 6], [17, 29, 16, 24]]
    sched = [(ks1, ks2), (ks2, ks0), (ks0, ks1), (ks1, ks2), (ks2, ks0)]
    for i in range(5):
        for r in rot[i % 2]:
            x0 = (x0 + x1).astype(np.uint32)
            x1 = rotl(x1, r)
            x1 = (x1 ^ x0).astype(np.uint32)
        a, b = sched[i]
        x0 = (x0 + a).astype(np.uint32)
        x1 = (x1 + b + np.uint32(i + 1)).astype(np.uint32)
    return x0, x1


def _random_bits(ks, n):
    x0, x1 = _threefry2x32(ks, np.zeros(n, np.uint32), np.arange(n, dtype=np.uint32))
    return (x0 ^ x1).astype(np.uint32)


def _sample_indices(L, sample_k, seed=42):
    key = (np.uint32(0), np.uint32(seed))
    x0, x1 = _threefry2x32(key, np.zeros(2, np.uint32), np.arange(2, dtype=np.uint32))
    k1, k2 = (x0[0], x1[0]), (x0[1], x1[1])
    n = L * sample_k
    hi, lo = _random_bits(k1, n), _random_bits(k2, n)
    span = np.uint32(L)
    mult = np.uint32((np.uint64(2) ** 16 % span) ** 2 % span)
    off = ((hi % span) * mult + lo % span) % span
    return off.astype(np.int32).reshape(L, sample_k)


@functools.lru_cache(maxsize=4)
def _constants(L, sample_k, CH):
    idx = _sample_indices(L, sample_k)
    AT = np.zeros((L, L), np.int32)
    np.add.at(AT, (idx, np.arange(L)[:, None]), 1)
    TRI = np.tril(np.ones((CH, CH), np.float32))
    return AT.astype(np.int8), TRI


def _east_body(x_ref, ste_ref, wq_ref, bq_ref, wk_ref, bk_ref, wv_ref, bv_ref,
               at_ref, tri_ref, out_ref, *, L, D, n_top, U, CH, HB):
    f32 = jnp.float32
    dn11 = (((1,), (1,)), ((), ()))
    dn10 = (((1,), (0,)), ((), ()))
    dn00 = (((0,), (0,)), ((), ()))

    def one_head(t):
        x = x_ref[0, t]
        s = ste_ref[0, t]

        def proj(w_ref, b_ref):
            w = w_ref[...]
            h = (lax.dot_general(x, w[:, :D], dn11)
                 + lax.dot_general(s, w[:, D:], dn11)
                 + b_ref[...])
            return jnp.maximum(h, 0.0)

        Qm = proj(wq_ref, bq_ref)
        Km = proj(wk_ref, bk_ref)
        Vm = proj(wv_ref, bv_ref)

        ST = lax.dot_general(Km, Qm, dn11)
        at8 = at_ref[...]
        atf = at8.astype(f32)
        smax = jnp.max(jnp.where(atf > 0.0, ST, -1e38), axis=0, keepdims=True)
        W = lax.dot_general(atf, Km, dn00)
        ssum_col = lax.dot_general(Qm * W, jnp.ones((D, 1), f32), dn10)
        ssum_row = jnp.transpose(ssum_col)
        Mrow = smax - ssum_row * (1.0 / L)
        Mcol = jnp.transpose(Mrow)

        ii = lax.broadcasted_iota(jnp.int32, (L, L), 0)
        jj = lax.broadcasted_iota(jnp.int32, (L, L), 1)
        beats = (Mcol > Mrow) | ((Mcol == Mrow) & (ii < jj))
        C = jnp.where(beats, 1.0, 0.0)
        rank = lax.dot_general(jnp.ones((1, L), f32), C, dn10)

        iota_uf = lax.broadcasted_iota(jnp.int32, (U, 1), 0).astype(f32)
        P = ((rank == iota_uf) & (iota_uf < n_top)).astype(f32)
        iota_cf = lax.broadcasted_iota(jnp.int32, (L, 1), 0).astype(f32)
        mt = lax.dot_general(P, iota_cf, dn10)

        lane = lax.broadcasted_iota(jnp.int32, (U, L), 1).astype(f32)
        Qr = lax.dot_general(P, Qm, dn10)
        sc = lax.dot_general(Qr, Km, dn11) * (1.0 / np.sqrt(D))
        sc = jnp.where(lane > mt, -1e30, sc)
        sc = sc - jnp.max(sc, axis=1, keepdims=True)
        e = jnp.exp(sc)
        attn = e / jnp.sum(e, axis=1, keepdims=True)
        upd = lax.dot_general(attn, Vm, dn10)

        tri = tri_ref[...]
        carry = jnp.zeros((1, D), f32)
        rows = []
        for i in range(L // CH):
            vi = lax.slice(Vm, (i * CH, 0), ((i + 1) * CH, D))
            rows.append(lax.dot_general(tri, vi, dn10) + carry)
            carry = carry + lax.dot_general(jnp.ones((1, CH), f32), vi, dn10)
        ctx = jnp.concatenate(rows, axis=0)

        scat = lax.dot_general(P, upd, dn00)
        selc = lax.dot_general(P, jnp.ones((U, 1), f32), dn00)
        return ctx * (1.0 - selc) + scat

    out_ref[0] = jnp.concatenate([one_head(t) for t in range(HB)], axis=1)


def kernel(X, STE, Wq, bq, Wk, bk, Wv, bv, attn_mask=0):
    B, H, L, D = X.shape
    n_top = min(_FACTOR * int(np.ceil(np.log(L))), L)
    U = ((n_top + 7) // 8) * 8
    CH = 128
    HB = 2
    AT8, TRI = _constants(L, n_top, CH)

    out = pl.pallas_call(
        functools.partial(_east_body, L=L, D=D, n_top=n_top, U=U, CH=CH, HB=HB),
        grid=(B, H // HB),
        in_specs=[
            pl.BlockSpec((1, HB, L, D), lambda b, h: (b, h, 0, 0)),
            pl.BlockSpec((1, HB, L, D), lambda b, h: (b, h, 0, 0)),
            pl.BlockSpec((D, 2 * D), lambda b, h: (0, 0)),
            pl.BlockSpec((1, D), lambda b, h: (0, 0)),
            pl.BlockSpec((D, 2 * D), lambda b, h: (0, 0)),
            pl.BlockSpec((1, D), lambda b, h: (0, 0)),
            pl.BlockSpec((D, 2 * D), lambda b, h: (0, 0)),
            pl.BlockSpec((1, D), lambda b, h: (0, 0)),
            pl.BlockSpec((L, L), lambda b, h: (0, 0)),
            pl.BlockSpec((CH, CH), lambda b, h: (0, 0)),
        ],
        out_specs=pl.BlockSpec((1, L, HB * D), lambda b, h: (b, 0, h)),
        out_shape=jax.ShapeDtypeStruct((B, L, H * D), jnp.float32),
        compiler_params=pltpu.CompilerParams(
            dimension_semantics=("parallel", "parallel")),
    )(X, STE, Wq, bq.reshape(1, D), Wk, bk.reshape(1, D), Wv, bv.reshape(1, D),
      jnp.asarray(AT8), jnp.asarray(TRI))
    return out.reshape(B, L, H, D)
